# baseline (speedup 1.0000x reference)
import functools

import jax
import jax.numpy as jnp
from jax import lax
from jax.experimental import pallas as pl
from jax.experimental.pallas import tpu as pltpu
from jax.experimental.pallas import tpu_sc as plsc

NC = 2
NS = 16
NW = NC * NS
L = 16

B = 16384
S = 50
D = 32
CB = B // NW
NPAIR = CB // 2
G = 2 * S
NBUF = 8
NOUTER = NPAIR // NBUF


def _tree_sum(vals):
    accs = list(vals[:4])
    for i in range(4, len(vals)):
        accs[i % 4] = accs[i % 4] + vals[i]
    return (accs[0] + accs[1]) + (accs[2] + accs[3])


def _sc_body(idx_hbm, props_hbm, table_hbm, w_hbm, b_hbm, out_hbm,
             idx_f, idx_v, props_v, ps_v, out_v, bufs, wv, bv, sems):
    wid = lax.axis_index("s") * NC + lax.axis_index("c")
    nwk = CB * S

    pltpu.sync_copy(idx_hbm.at[pl.ds(wid * nwk, nwk)], idx_f)

    def repack(j, carry):
        base = G * j
        for t in range(6):
            idx_v[j, pl.ds(t * L, L)] = idx_f[pl.ds(base + t * L, L)]
        idx_v[j, pl.ds(G - L, L)] = idx_f[pl.ds(base + G - L, L)]
        return carry

    lax.fori_loop(0, NPAIR, repack, 0)

    def start(j, buf, sem):
        pltpu.make_async_copy(table_hbm.at[idx_v.at[j]], buf, sem).start()

    for bi in range(NBUF):
        start(bi, bufs[bi], sems[bi])

    pltpu.sync_copy(props_hbm.at[pl.ds(wid * nwk, nwk)], props_v)
    pltpu.sync_copy(w_hbm, wv)
    pltpu.sync_copy(b_hbm, bv)

    iota = lax.iota(jnp.int32, L)
    w_h = [wv[pl.ds(0, L)], wv[pl.ds(L, L)]]
    sb_h = [bv[pl.ds(0, L)] * float(S), bv[pl.ds(L, L)] * float(S)]
    tail_mask = iota >= 4 * L - S
    fzero = jnp.zeros((L,), jnp.float32)

    def ps_body(j, carry):
        sums = []
        for r in range(2):
            rb = S * (2 * j + r)
            acc = (props_v[pl.ds(rb, L)] + props_v[pl.ds(rb + L, L)]
                   + props_v[pl.ds(rb + 2 * L, L)])
            acc = acc + jnp.where(tail_mask,
                                  props_v[pl.ds(rb + S - L, L)], fzero)
            for k in (8, 4, 2, 1):
                acc = acc + acc[(iota + k) & (L - 1)]
            sums.append(acc)
        ps_v[j, :] = jnp.where(iota < 1, sums[0], sums[1])
        return carry

    lax.fori_loop(0, NPAIR, ps_body, 0)

    def accum(j, buf):
        ps_vec = ps_v[j, :]
        for r in range(2):
            row = 2 * j + r
            ps_s = ps_vec[r]
            for h in range(2):
                tot = _tree_sum(
                    [buf[r * S + i, pl.ds(h * L, L)] for i in range(S)])
                out_v[row, pl.ds(h * L, L)] = tot + ps_s * w_h[h] + sb_h[h]

    def body(i, carry):
        for bi in range(NBUF):
            j = i * NBUF + bi
            pltpu.make_async_copy(
                table_hbm.at[idx_v.at[j]], bufs[bi], sems[bi]).wait()
            accum(j, bufs[bi])

            @pl.when(i < NOUTER - 1)
            def _():
                start(j + NBUF, bufs[bi], sems[bi])
        return carry

    lax.fori_loop(0, NOUTER, body, 0)

    pltpu.sync_copy(out_v, out_hbm.at[pl.ds(wid * CB, CB)])


@functools.lru_cache(maxsize=1)
def _make_sc_kernel():
    @functools.partial(
        pl.kernel,
        out_type=jax.ShapeDtypeStruct((B, D), jnp.float32),
        mesh=plsc.VectorSubcoreMesh(core_axis_name="c", subcore_axis_name="s",
                                    num_cores=NC, num_subcores=NS),
        compiler_params=pltpu.CompilerParams(use_tc_tiling_on_sc=False),
        scratch_types=dict(
            idx_f=pltpu.VMEM((CB * S,), jnp.int32),
            idx_v=pltpu.VMEM((NPAIR, G), jnp.int32),
            props_v=pltpu.VMEM((CB * S,), jnp.float32),
            ps_v=pltpu.VMEM((NPAIR, L), jnp.float32),
            out_v=pltpu.VMEM((CB, D), jnp.float32),
            bufs=[pltpu.VMEM((G, D), jnp.float32) for _ in range(NBUF)],
            wv=pltpu.VMEM((D,), jnp.float32),
            bv=pltpu.VMEM((D,), jnp.float32),
            sems=[pltpu.SemaphoreType.DMA for _ in range(NBUF)],
        ),
    )
    def _sc_kernel(idx_hbm, props_hbm, table_hbm, w_hbm, b_hbm, out_hbm,
                   idx_f, idx_v, props_v, ps_v, out_v, bufs, wv, bv, sems):
        _sc_body(idx_hbm, props_hbm, table_hbm, w_hbm, b_hbm, out_hbm,
                 idx_f, idx_v, props_v, ps_v, out_v, bufs, wv, bv, sems)

    return _sc_kernel


def kernel(x, table, W, b):
    idx = x[..., 0].astype(jnp.int32).reshape(B * S)
    props = x[..., 1].reshape(B * S)
    w = W[:, 0]
    return _make_sc_kernel()(idx, props, table, w, b)


# speedup vs baseline: 1.1544x; 1.1544x over previous
import functools

import jax
import jax.numpy as jnp
from jax import lax
from jax.experimental import pallas as pl
from jax.experimental.pallas import tpu as pltpu
from jax.experimental.pallas import tpu_sc as plsc

NC = 2
NS = 16
NW = NC * NS
L = 16

B = 16384
S = 50
D = 32
CB = B // NW
NPAIR = CB // 2
G = 2 * S
NBUF = 4
NOUTER = NPAIR // NBUF


def _tree_sum(vals):
    accs = list(vals[:4])
    for i in range(4, len(vals)):
        accs[i % 4] = accs[i % 4] + vals[i]
    return (accs[0] + accs[1]) + (accs[2] + accs[3])


def _sc_body(idx_hbm, props_hbm, table_hbm, w_hbm, b_hbm, out_hbm,
             idx_f, idx_v, props_v, ps_v, out_v, bufs, wv, bv, sems):
    wid = lax.axis_index("s") * NC + lax.axis_index("c")
    nwk = CB * S

    pltpu.sync_copy(idx_hbm.at[pl.ds(wid * nwk, nwk)], idx_f)

    def repack(j, carry):
        base = G * j
        for t in range(6):
            idx_v[j, pl.ds(t * L, L)] = idx_f[pl.ds(base + t * L, L)]
        idx_v[j, pl.ds(G - L, L)] = idx_f[pl.ds(base + G - L, L)]
        return carry

    lax.fori_loop(0, NPAIR, repack, 0)

    def start(j, buf, sem):
        pltpu.make_async_copy(table_hbm.at[idx_v.at[j]], buf, sem).start()

    for bi in range(NBUF):
        start(bi, bufs[bi], sems[bi])

    pltpu.sync_copy(props_hbm.at[pl.ds(wid * nwk, nwk)], props_v)
    pltpu.sync_copy(w_hbm, wv)
    pltpu.sync_copy(b_hbm, bv)

    iota = lax.iota(jnp.int32, L)
    w_h = [wv[pl.ds(0, L)], wv[pl.ds(L, L)]]
    sb_h = [bv[pl.ds(0, L)] * float(S), bv[pl.ds(L, L)] * float(S)]
    tail_mask = iota >= 4 * L - S
    fzero = jnp.zeros((L,), jnp.float32)

    def ps_body(j, carry):
        sums = []
        for r in range(2):
            rb = S * (2 * j + r)
            acc = (props_v[pl.ds(rb, L)] + props_v[pl.ds(rb + L, L)]
                   + props_v[pl.ds(rb + 2 * L, L)])
            acc = acc + jnp.where(tail_mask,
                                  props_v[pl.ds(rb + S - L, L)], fzero)
            for k in (8, 4, 2, 1):
                acc = acc + acc[(iota + k) & (L - 1)]
            sums.append(acc)
        ps_v[j, :] = jnp.where(iota < 1, sums[0], sums[1])
        return carry

    lax.fori_loop(0, NPAIR, ps_body, 0)

    def accum(j, buf):
        ps_vec = ps_v[j, :]
        for r in range(2):
            row = 2 * j + r
            ps_s = ps_vec[r]
            for h in range(2):
                tot = _tree_sum(
                    [buf[r * S + i, pl.ds(h * L, L)] for i in range(S)])
                out_v[row, pl.ds(h * L, L)] = tot + ps_s * w_h[h] + sb_h[h]

    def body(i, carry):
        for bi in range(NBUF):
            j = i * NBUF + bi
            pltpu.make_async_copy(
                table_hbm.at[idx_v.at[j]], bufs[bi], sems[bi]).wait()
            accum(j, bufs[bi])

            @pl.when(i < NOUTER - 1)
            def _():
                start(j + NBUF, bufs[bi], sems[bi])
        return carry

    lax.fori_loop(0, NOUTER, body, 0)

    pltpu.sync_copy(out_v, out_hbm.at[pl.ds(wid * CB, CB)])


@functools.lru_cache(maxsize=1)
def _make_sc_kernel():
    @functools.partial(
        pl.kernel,
        out_type=jax.ShapeDtypeStruct((B, D), jnp.float32),
        mesh=plsc.VectorSubcoreMesh(core_axis_name="c", subcore_axis_name="s",
                                    num_cores=NC, num_subcores=NS),
        compiler_params=pltpu.CompilerParams(use_tc_tiling_on_sc=False),
        scratch_types=dict(
            idx_f=pltpu.VMEM((CB * S,), jnp.int32),
            idx_v=pltpu.VMEM((NPAIR, G), jnp.int32),
            props_v=pltpu.VMEM((CB * S,), jnp.float32),
            ps_v=pltpu.VMEM((NPAIR, L), jnp.float32),
            out_v=pltpu.VMEM((CB, D), jnp.float32),
            bufs=[pltpu.VMEM((G, D), jnp.float32) for _ in range(NBUF)],
            wv=pltpu.VMEM((D,), jnp.float32),
            bv=pltpu.VMEM((D,), jnp.float32),
            sems=[pltpu.SemaphoreType.DMA for _ in range(NBUF)],
        ),
    )
    def _sc_kernel(idx_hbm, props_hbm, table_hbm, w_hbm, b_hbm, out_hbm,
                   idx_f, idx_v, props_v, ps_v, out_v, bufs, wv, bv, sems):
        _sc_body(idx_hbm, props_hbm, table_hbm, w_hbm, b_hbm, out_hbm,
                 idx_f, idx_v, props_v, ps_v, out_v, bufs, wv, bv, sems)

    return _sc_kernel


def kernel(x, table, W, b):
    idx = x[..., 0].astype(jnp.int32).reshape(B * S)
    props = x[..., 1].reshape(B * S)
    w = W[:, 0]
    return _make_sc_kernel()(idx, props, table, w, b)
